# Initial kernel scaffold; baseline (speedup 1.0000x reference)
import jax
import jax.numpy as jnp
from jax.experimental import pallas as pl


def kernel(k_cache, v_cache, new_k, new_v, attention_mask, position_ids):
    raise NotImplementedError("write your pallas kernel here")


# speedup vs baseline: 1.0119x; 1.0119x over previous
import jax
import jax.numpy as jnp
from jax.experimental import pallas as pl
from jax.experimental.pallas import tpu as pltpu

_L, _B, _H, _S, _D = 8, 2, 8, 4096, 128
_F = _L * _B * _H
_R = 2


def _scatter_body(pos_ref, k_ref, v_ref, nk_ref, nv_ref, ko_ref, vo_ref):
    i = pl.program_id(0)
    b = (i * _R // _H) % _B
    pos = pos_ref[b]
    sel = jax.lax.broadcasted_iota(jnp.int32, (1, _S, 1), 1) == pos
    ko_ref[...] = jnp.where(sel, nk_ref[...], k_ref[...])
    vo_ref[...] = jnp.where(sel, nv_ref[...], v_ref[...])


def kernel(k_cache, v_cache, new_k, new_v, attention_mask, position_ids):
    mask = attention_mask[:, None, None, :].astype(bool)

    kf = k_cache.reshape(_F, _S, _D)
    vf = v_cache.reshape(_F, _S, _D)
    nk = new_k.reshape(_F, 1, _D)
    nv = new_v.reshape(_F, 1, _D)
    pos = position_ids.reshape(_B)

    big = pl.BlockSpec((_R, _S, _D), lambda i, pos_ref: (i, 0, 0))
    row = pl.BlockSpec((_R, 1, _D), lambda i, pos_ref: (i, 0, 0))

    grid_spec = pltpu.PrefetchScalarGridSpec(
        num_scalar_prefetch=1,
        grid=(_F // _R,),
        in_specs=[big, big, row, row],
        out_specs=[big, big],
    )
    ko, vo = pl.pallas_call(
        _scatter_body,
        grid_spec=grid_spec,
        out_shape=[jax.ShapeDtypeStruct((_F, _S, _D), k_cache.dtype)] * 2,
        compiler_params=pltpu.CompilerParams(
            dimension_semantics=("parallel",),
            vmem_limit_bytes=48 * 1024 * 1024,
        ),
    )(pos, kf, vf, nk, nv)

    return (
        mask,
        ko.reshape(_L, _B, _H, _S, _D),
        vo.reshape(_L, _B, _H, _S, _D),
    )
